# baseline (speedup 1.0000x reference)
import functools

import jax
import jax.numpy as jnp
from jax import lax
from jax.experimental import pallas as pl
from jax.experimental.pallas import tpu as pltpu
from jax.experimental.pallas import tpu_sc as plsc

B = 16384
F = 26
V = 100000
D = 64

_info = plsc.get_sparse_core_info()
NC, NS, L = _info.num_cores, _info.num_subcores, _info.num_lanes
NW = NC * NS
UNITS = F * D
UPW = UNITS // NW
NQ = 4
BQ = B // NQ


def _sc_embedding(spT_hbm, tt_hbm, out_hbm, src_v, idx_v, outA_v, outB_v,
                  semA, semB):
    wid = lax.axis_index("s") * NC + lax.axis_index("c")
    obufs = (outA_v, semA), (outB_v, semB)

    def unit_body(j, carry):
        prev_f, flag = carry
        u = wid * UPW + j
        f = u // D
        d = u % D

        @pl.when(f != prev_f)
        def _load_idx():
            pltpu.sync_copy(spT_hbm.at[f, :], idx_v)

        def new_flag():
            def acc_body(k, acc):
                return acc + idx_v[pl.ds(k * L, L)]
            acc = lax.fori_loop(0, B // L, acc_body,
                                jnp.zeros((L,), jnp.int32))
            s = acc[0]
            for l in range(1, L):
                s = s + acc[l]
            return (s == 0).astype(jnp.int32)

        flag = lax.cond(f != prev_f, new_flag, lambda: flag)

        pltpu.sync_copy(tt_hbm.at[f, d, :], src_v)

        for q in range(NQ):
            ov, sm = obufs[q % 2]

            def _drain():
                pltpu.make_async_copy(
                    out_hbm.at[0, 0, pl.ds(0, BQ)], ov, sm).wait()

            if q >= 2:
                _drain()
            else:
                pl.when(j > 0)(_drain)

            @pl.when(flag == 0)
            def _gather():
                def g_body(k, carry2):
                    base = k * (8 * L)
                    for t in range(8):
                        sl = pl.ds(base + t * L, L)
                        iv = idx_v[pl.ds(q * BQ + base + t * L, L)]
                        ov[sl] = plsc.load_gather(src_v, [iv])
                    return carry2
                lax.fori_loop(0, BQ // (8 * L), g_body, 0)

            @pl.when(flag == 1)
            def _ones():
                ones_l = jnp.ones((L,), jnp.float32)

                def o_body(k, carry2):
                    base = k * (8 * L)
                    for t in range(8):
                        ov[pl.ds(base + t * L, L)] = ones_l
                    return carry2
                lax.fori_loop(0, BQ // (8 * L), o_body, 0)

            pltpu.async_copy(ov, out_hbm.at[f, d, pl.ds(q * BQ, BQ)], sm)

        return (f, flag)

    lax.fori_loop(0, UPW, unit_body, (jnp.int32(-1), jnp.int32(0)))
    for ov, sm in obufs:
        pltpu.make_async_copy(out_hbm.at[0, 0, pl.ds(0, BQ)], ov, sm).wait()


@jax.jit
def kernel(sparse_inputs, tables, fixed_vector):
    del fixed_vector
    spT = sparse_inputs.T
    tt = jnp.transpose(tables, (0, 2, 1))

    run = functools.partial(
        pl.kernel,
        mesh=plsc.VectorSubcoreMesh(core_axis_name="c", subcore_axis_name="s"),
        out_type=jax.ShapeDtypeStruct((F, D, B), jnp.float32),
        compiler_params=pltpu.CompilerParams(use_tc_tiling_on_sc=True,
                                             needs_layout_passes=False),
        scratch_types=[
            pltpu.VMEM((V,), jnp.float32),
            pltpu.VMEM((B,), jnp.int32),
            pltpu.VMEM((BQ,), jnp.float32),
            pltpu.VMEM((BQ,), jnp.float32),
            pltpu.SemaphoreType.DMA,
            pltpu.SemaphoreType.DMA,
        ],
    )(_sc_embedding)

    outp = run(spT, tt)
    return jnp.transpose(outp, (2, 0, 1))


# speedup vs baseline: 1.0204x; 1.0204x over previous
import functools

import jax
import jax.numpy as jnp
from jax import lax
from jax.experimental import pallas as pl
from jax.experimental.pallas import tpu as pltpu
from jax.experimental.pallas import tpu_sc as plsc

B = 16384
F = 26
V = 100000
D = 64

_info = plsc.get_sparse_core_info()
NC, NS, L = _info.num_cores, _info.num_subcores, _info.num_lanes
NW = NC * NS
UNITS = F * D
UPW = UNITS // NW
NQ = 4
BQ = B // NQ


def _sc_embedding(spT_hbm, tt_hbm, out_hbm, src_v, idx_v, outA_v, outB_v,
                  semA, semB):
    wid = lax.axis_index("s") * NC + lax.axis_index("c")
    obufs = (outA_v, semA), (outB_v, semB)

    def unit_body(j, carry):
        prev_f, flag = carry
        u = wid * UPW + j
        f = u // D
        d = u % D

        @pl.when(f != prev_f)
        def _load_idx():
            pltpu.sync_copy(spT_hbm.at[f, :], idx_v)

        def new_flag():
            def acc_body(k, acc):
                base = k * (8 * L)
                for t in range(8):
                    acc = acc + idx_v[pl.ds(base + t * L, L)]
                return acc
            acc = lax.fori_loop(0, B // (8 * L), acc_body,
                                jnp.zeros((L,), jnp.int32))
            s = acc[0]
            for l in range(1, L):
                s = s + acc[l]
            return (s == 0).astype(jnp.int32)

        flag = lax.cond(f != prev_f, new_flag, lambda: flag)

        pltpu.sync_copy(tt_hbm.at[f, d, :], src_v)

        for q in range(NQ):
            ov, sm = obufs[q % 2]

            def _drain():
                pltpu.make_async_copy(
                    out_hbm.at[0, 0, pl.ds(0, BQ)], ov, sm).wait()

            if q >= 2:
                _drain()
            else:
                pl.when(j > 0)(_drain)

            @pl.when(flag == 0)
            def _gather():
                def g_body(k, carry2):
                    base = k * (16 * L)
                    for t in range(16):
                        sl = pl.ds(base + t * L, L)
                        iv = idx_v[pl.ds(q * BQ + base + t * L, L)]
                        ov[sl] = plsc.load_gather(src_v, [iv])
                    return carry2
                lax.fori_loop(0, BQ // (16 * L), g_body, 0)

            @pl.when(flag == 1)
            def _ones():
                ones_l = jnp.ones((L,), jnp.float32)

                def o_body(k, carry2):
                    base = k * (8 * L)
                    for t in range(8):
                        ov[pl.ds(base + t * L, L)] = ones_l
                    return carry2
                lax.fori_loop(0, BQ // (8 * L), o_body, 0)

            pltpu.async_copy(ov, out_hbm.at[f, d, pl.ds(q * BQ, BQ)], sm)

        return (f, flag)

    lax.fori_loop(0, UPW, unit_body, (jnp.int32(-1), jnp.int32(0)))
    for ov, sm in obufs:
        pltpu.make_async_copy(out_hbm.at[0, 0, pl.ds(0, BQ)], ov, sm).wait()


@jax.jit
def kernel(sparse_inputs, tables, fixed_vector):
    del fixed_vector
    spT = sparse_inputs.T
    tt = jnp.transpose(tables, (0, 2, 1))

    run = functools.partial(
        pl.kernel,
        mesh=plsc.VectorSubcoreMesh(core_axis_name="c", subcore_axis_name="s"),
        out_type=jax.ShapeDtypeStruct((F, D, B), jnp.float32),
        compiler_params=pltpu.CompilerParams(use_tc_tiling_on_sc=True,
                                             needs_layout_passes=False),
        scratch_types=[
            pltpu.VMEM((V,), jnp.float32),
            pltpu.VMEM((B,), jnp.int32),
            pltpu.VMEM((BQ,), jnp.float32),
            pltpu.VMEM((BQ,), jnp.float32),
            pltpu.SemaphoreType.DMA,
            pltpu.SemaphoreType.DMA,
        ],
    )(_sc_embedding)

    outp = run(spT, tt)
    return jnp.transpose(outp, (2, 0, 1))


# speedup vs baseline: 1.0218x; 1.0014x over previous
import functools

import jax
import jax.numpy as jnp
from jax import lax
from jax.experimental import pallas as pl
from jax.experimental.pallas import tpu as pltpu
from jax.experimental.pallas import tpu_sc as plsc

B = 16384
F = 26
V = 100000
D = 64

_info = plsc.get_sparse_core_info()
NC, NS, L = _info.num_cores, _info.num_subcores, _info.num_lanes
NW = NC * NS
UNITS = F * D
UPW = UNITS // NW
NQ = 4
BQ = B // NQ


def _sc_embedding(spT_hbm, tt_hbm, out_hbm, src_v, idx_v, outA_v, outB_v,
                  semA, semB):
    wid = lax.axis_index("s") * NC + lax.axis_index("c")
    obufs = (outA_v, semA), (outB_v, semB)

    def unit_body(j, carry):
        prev_f, flag = carry
        u = wid * UPW + j
        f = u // D
        d = u % D

        @pl.when(f != prev_f)
        def _load_idx():
            pltpu.sync_copy(spT_hbm.at[f, :], idx_v)

        def new_flag():
            def acc_body(k, acc):
                base = k * (8 * L)
                for t in range(8):
                    acc = acc + idx_v[pl.ds(base + t * L, L)]
                return acc
            acc = lax.fori_loop(0, B // (8 * L), acc_body,
                                jnp.zeros((L,), jnp.int32))
            s = acc[0]
            for l in range(1, L):
                s = s + acc[l]
            return (s == 0).astype(jnp.int32)

        flag = lax.cond(f != prev_f, new_flag, lambda: flag)

        pltpu.sync_copy(tt_hbm.at[f, d, :], src_v)

        for q in range(NQ):
            ov, sm = obufs[q % 2]

            def _drain():
                pltpu.make_async_copy(
                    out_hbm.at[0, 0, pl.ds(0, BQ)], ov, sm).wait()

            if q >= 2:
                _drain()
            else:
                pl.when(j > 0)(_drain)

            @pl.when(flag == 0)
            def _gather():
                def g_body(k, carry2):
                    base = k * (32 * L)
                    for t in range(32):
                        sl = pl.ds(base + t * L, L)
                        iv = idx_v[pl.ds(q * BQ + base + t * L, L)]
                        ov[sl] = plsc.load_gather(src_v, [iv])
                    return carry2
                lax.fori_loop(0, BQ // (32 * L), g_body, 0)

            @pl.when(flag == 1)
            def _ones():
                ones_l = jnp.ones((L,), jnp.float32)

                def o_body(k, carry2):
                    base = k * (8 * L)
                    for t in range(8):
                        ov[pl.ds(base + t * L, L)] = ones_l
                    return carry2
                lax.fori_loop(0, BQ // (8 * L), o_body, 0)

            pltpu.async_copy(ov, out_hbm.at[f, d, pl.ds(q * BQ, BQ)], sm)

        return (f, flag)

    lax.fori_loop(0, UPW, unit_body, (jnp.int32(-1), jnp.int32(0)))
    for ov, sm in obufs:
        pltpu.make_async_copy(out_hbm.at[0, 0, pl.ds(0, BQ)], ov, sm).wait()


@jax.jit
def kernel(sparse_inputs, tables, fixed_vector):
    del fixed_vector
    spT = sparse_inputs.T
    tt = jnp.transpose(tables, (0, 2, 1))

    run = functools.partial(
        pl.kernel,
        mesh=plsc.VectorSubcoreMesh(core_axis_name="c", subcore_axis_name="s"),
        out_type=jax.ShapeDtypeStruct((F, D, B), jnp.float32),
        compiler_params=pltpu.CompilerParams(use_tc_tiling_on_sc=True,
                                             needs_layout_passes=False),
        scratch_types=[
            pltpu.VMEM((V,), jnp.float32),
            pltpu.VMEM((B,), jnp.int32),
            pltpu.VMEM((BQ,), jnp.float32),
            pltpu.VMEM((BQ,), jnp.float32),
            pltpu.SemaphoreType.DMA,
            pltpu.SemaphoreType.DMA,
        ],
    )(_sc_embedding)

    outp = run(spT, tt)
    return jnp.transpose(outp, (2, 0, 1))
